# baseline (speedup 1.0000x reference)
import functools

import jax
import jax.numpy as jnp
from jax import lax
from jax.experimental import pallas as pl
from jax.experimental.pallas import tpu as pltpu
from jax.experimental.pallas import tpu_sc as plsc

_B = 16384
_K = 20
_D = 64
_NW = 32
_N_PER_W = _B // _NW
_ICHUNK = 4
_NCHUNKS = _N_PER_W // _ICHUNK
_ROWS = _ICHUNK * _K


def _sc_scores(pos_u_r, pos_v_r, neg_r, u_weight, v_weight):
    mesh = plsc.VectorSubcoreMesh(core_axis_name="c", subcore_axis_name="s")

    @functools.partial(
        pl.kernel,
        mesh=mesh,
        compiler_params=pltpu.CompilerParams(use_tc_tiling_on_sc=False),
        out_type=[
            jax.ShapeDtypeStruct((_NW, _NCHUNKS * 16), jnp.float32),
            jax.ShapeDtypeStruct((_NW, _N_PER_W * _K), jnp.float32),
        ],
        scratch_types=[
            pltpu.VMEM((_ICHUNK, _NCHUNKS), jnp.int32),
            pltpu.VMEM((_ICHUNK, _NCHUNKS), jnp.int32),
            pltpu.VMEM((_NCHUNKS, _ROWS), jnp.int32),
            pltpu.VMEM((_N_PER_W, _D), jnp.float32),
            pltpu.VMEM((_N_PER_W, _D), jnp.float32),
            pltpu.VMEM((_ROWS, _D), jnp.float32),
            pltpu.VMEM((_ROWS, _D), jnp.float32),
            pltpu.VMEM((_NCHUNKS * 16,), jnp.float32),
            pltpu.VMEM((_N_PER_W * _K,), jnp.float32),
            pltpu.SemaphoreType.DMA,
            pltpu.SemaphoreType.DMA,
            pltpu.SemaphoreType.DMA,
            pltpu.SemaphoreType.DMA,
        ],
    )
    def scores_kernel(pu_hbm, pv_hbm, nv_hbm, uw_hbm, vw_hbm,
                      pos_out, neg_out,
                      pu_i, pv_i, nv_i, u_rows, v_rows, nb0, nb1,
                      pos_v, neg_sv, sem_u, sem_v, sem0, sem1):
        wid = lax.axis_index("s") * 2 + lax.axis_index("c")
        lane = lax.iota(jnp.int32, 16)
        last = lane == 15
        perms = [lane ^ sh for sh in (8, 4, 2, 1)]

        def xsum(x):
            for p in perms:
                x = x + x.at[p].get(mode="promise_in_bounds")
            return x

        pltpu.sync_copy(pu_hbm.at[wid], pu_i)
        pltpu.sync_copy(pv_hbm.at[wid], pv_i)
        pltpu.sync_copy(nv_hbm.at[wid], nv_i)

        for c in range(_ICHUNK):
            pltpu.async_copy(uw_hbm.at[pu_i.at[c]],
                             u_rows.at[pl.ds(c * _NCHUNKS, _NCHUNKS)], sem_u)
            pltpu.async_copy(vw_hbm.at[pv_i.at[c]],
                             v_rows.at[pl.ds(c * _NCHUNKS, _NCHUNKS)], sem_v)

        def fire_neg(c, buf, sem):
            pltpu.async_copy(vw_hbm.at[nv_i.at[c]], buf, sem)

        def wait_neg(c, buf, sem):
            pltpu.make_async_copy(vw_hbm.at[nv_i.at[c]], buf, sem).wait()

        fire_neg(0, nb0, sem0)
        fire_neg(1, nb1, sem1)

        for c in range(_ICHUNK):
            pltpu.make_async_copy(
                uw_hbm.at[pu_i.at[c]],
                u_rows.at[pl.ds(c * _NCHUNKS, _NCHUNKS)], sem_u).wait()
            pltpu.make_async_copy(
                vw_hbm.at[pv_i.at[c]],
                v_rows.at[pl.ds(c * _NCHUNKS, _NCHUNKS)], sem_v).wait()

        def compute_chunk(c, buf):
            rpos = jnp.zeros((16,), jnp.float32)
            rneg = [jnp.zeros((16,), jnp.float32) for _ in range(5)]
            for ii in range(_ICHUNK):
                i = c * _ICHUNK + ii
                u = [u_rows[i, pl.ds(16 * j, 16)] for j in range(4)]
                v = [v_rows[i, pl.ds(16 * j, 16)] for j in range(4)]
                s = u[0] * v[0] + u[1] * v[1] + u[2] * v[2] + u[3] * v[3]
                rpos = jnp.where(lane == ii, xsum(s), rpos)
                for k in range(_K):
                    g = ii * _K + k
                    n = [buf[g, pl.ds(16 * j, 16)] for j in range(4)]
                    s = u[0] * n[0] + u[1] * n[1] + u[2] * n[2] + u[3] * n[3]
                    rneg[g // 16] = jnp.where(lane == (g % 16), xsum(s),
                                              rneg[g // 16])
            pos_v[pl.ds(c * 16, 16)] = rpos
            for j in range(5):
                neg_sv[pl.ds(c * _ROWS + j * 16, 16)] = rneg[j]

        def body(t, carry):
            c0 = 2 * t
            wait_neg(c0, nb0, sem0)

            @pl.when(t <= (_NCHUNKS // 2) - 2)
            def _():
                fire_neg(c0 + 2, nb0, sem0)

            compute_chunk(c0, nb0)

            c1 = 2 * t + 1
            wait_neg(c1, nb1, sem1)

            @pl.when(t <= (_NCHUNKS // 2) - 2)
            def _():
                fire_neg(c1 + 2, nb1, sem1)

            compute_chunk(c1, nb1)
            return carry

        lax.fori_loop(0, _NCHUNKS // 2, body, 0)

        pltpu.sync_copy(pos_v, pos_out.at[wid])
        pltpu.sync_copy(neg_sv, neg_out.at[wid])

    return scores_kernel(pos_u_r, pos_v_r, neg_r, u_weight, v_weight)


def _transpose_body(in_ref, out_ref):
    out_ref[...] = in_ref[...].T


def _to_row_major(wt):
    v = wt.shape[1]
    n = 4096
    grid = (v + n - 1) // n
    return pl.pallas_call(
        _transpose_body,
        grid=(grid,),
        in_specs=[pl.BlockSpec((_D, n), lambda g: (0, g))],
        out_specs=pl.BlockSpec((n, _D), lambda g: (g, 0)),
        out_shape=jax.ShapeDtypeStruct((v, _D), jnp.float32),
    )(wt)


def _loss_body(pos_ref, neg_ref, out_ref):
    p = pos_ref[...]
    n = neg_ref[...]
    valid = lax.broadcasted_iota(jnp.int32, p.shape, 1) % 16 < _ICHUNK
    lp = jnp.minimum(p, 0.0) - jnp.log1p(jnp.exp(-jnp.abs(p)))
    lp = jnp.where(valid, lp, 0.0)
    ln = jnp.minimum(-n, 0.0) - jnp.log1p(jnp.exp(-jnp.abs(n)))
    out_ref[0, 0] = -(jnp.sum(lp) + jnp.sum(ln))


def kernel(pos_u, pos_v, neg_v, u_weight, v_weight):
    pu = pos_u.reshape(_NW, _ICHUNK, _NCHUNKS).astype(jnp.int32)
    pv = pos_v.reshape(_NW, _ICHUNK, _NCHUNKS).astype(jnp.int32)
    nv = neg_v.reshape(_NW, _NCHUNKS, _ROWS).astype(jnp.int32)

    u_rm = _to_row_major(u_weight.T)
    v_rm = _to_row_major(v_weight.T)
    pos_s, neg_s = _sc_scores(pu, pv, nv, u_rm, v_rm)

    loss = pl.pallas_call(
        _loss_body,
        out_shape=jax.ShapeDtypeStruct((1, 1), jnp.float32),
        out_specs=pl.BlockSpec(memory_space=pltpu.SMEM),
    )(pos_s.reshape(512, 128), neg_s.reshape(2560, 128))
    return loss[0, 0]


# speedup vs baseline: 2.3897x; 2.3897x over previous
import functools

import jax
import jax.numpy as jnp
from jax import lax
from jax.experimental import pallas as pl
from jax.experimental.pallas import tpu as pltpu
from jax.experimental.pallas import tpu_sc as plsc

_B = 16384
_K = 20
_D = 64
_NW = 32
_N_PER_W = _B // _NW
_ICHUNK = 4
_NCHUNKS = _N_PER_W // _ICHUNK
_ROWS = _ICHUNK * _K
_GROWS = 2 * _ICHUNK + _ROWS


def _sc_scores(idx_r, cw):
    mesh = plsc.VectorSubcoreMesh(core_axis_name="c", subcore_axis_name="s")

    @functools.partial(
        pl.kernel,
        mesh=mesh,
        compiler_params=pltpu.CompilerParams(use_tc_tiling_on_sc=False),
        out_type=[
            jax.ShapeDtypeStruct((_NW, _NCHUNKS * 16), jnp.float32),
            jax.ShapeDtypeStruct((_NW, _N_PER_W * _K), jnp.float32),
        ],
        scratch_types=[
            pltpu.VMEM((_NCHUNKS, _GROWS), jnp.int32),
            pltpu.VMEM((_GROWS, 2 * _D), jnp.float32),
            pltpu.VMEM((_GROWS, 2 * _D), jnp.float32),
            pltpu.VMEM((_NCHUNKS * 16,), jnp.float32),
            pltpu.VMEM((_N_PER_W * _K,), jnp.float32),
            pltpu.SemaphoreType.DMA,
            pltpu.SemaphoreType.DMA,
        ],
    )
    def scores_kernel(idx_hbm, cw_hbm, pos_out, neg_out,
                      idx_v, nb0, nb1, pos_v, neg_sv, sem0, sem1):
        wid = lax.axis_index("s") * 2 + lax.axis_index("c")
        lane = lax.iota(jnp.int32, 16)
        perms = [lane ^ sh for sh in (8, 4, 2, 1)]

        def xsum(x):
            for p in perms:
                x = x + x.at[p].get(mode="promise_in_bounds")
            return x

        pltpu.sync_copy(idx_hbm.at[wid], idx_v)

        def fire(c, buf, sem):
            pltpu.async_copy(cw_hbm.at[idx_v.at[c]], buf, sem)

        def wait(c, buf, sem):
            pltpu.make_async_copy(cw_hbm.at[idx_v.at[c]], buf, sem).wait()

        fire(0, nb0, sem0)
        fire(1, nb1, sem1)

        def compute_chunk(c, buf):
            rpos = jnp.zeros((16,), jnp.float32)
            rneg = [jnp.zeros((16,), jnp.float32) for _ in range(5)]
            for ii in range(_ICHUNK):
                u = [buf[ii, pl.ds(16 * j, 16)] for j in range(4)]
                v = [buf[4 + ii, pl.ds(_D + 16 * j, 16)] for j in range(4)]
                s = u[0] * v[0] + u[1] * v[1] + u[2] * v[2] + u[3] * v[3]
                rpos = jnp.where(lane == ii, xsum(s), rpos)
                for k in range(_K):
                    g = ii * _K + k
                    n = [buf[8 + g, pl.ds(_D + 16 * j, 16)] for j in range(4)]
                    s = u[0] * n[0] + u[1] * n[1] + u[2] * n[2] + u[3] * n[3]
                    rneg[g // 16] = jnp.where(lane == (g % 16), xsum(s),
                                              rneg[g // 16])
            pos_v[pl.ds(c * 16, 16)] = rpos
            for j in range(5):
                neg_sv[pl.ds(c * _ROWS + j * 16, 16)] = rneg[j]

        def body(t, carry):
            c0 = 2 * t
            wait(c0, nb0, sem0)

            @pl.when(t <= (_NCHUNKS // 2) - 2)
            def _():
                fire(c0 + 2, nb0, sem0)

            compute_chunk(c0, nb0)

            c1 = 2 * t + 1
            wait(c1, nb1, sem1)

            @pl.when(t <= (_NCHUNKS // 2) - 2)
            def _():
                fire(c1 + 2, nb1, sem1)

            compute_chunk(c1, nb1)
            return carry

        lax.fori_loop(0, _NCHUNKS // 2, body, 0)

        pltpu.sync_copy(pos_v, pos_out.at[wid])
        pltpu.sync_copy(neg_sv, neg_out.at[wid])

    return scores_kernel(idx_r, cw)


def _transpose_body(u_ref, v_ref, out_ref):
    out_ref[...] = jnp.concatenate([u_ref[...].T, v_ref[...].T], axis=1)


def _combined_table(ut, vt):
    v = ut.shape[1]
    n = 4096
    grid = (v + n - 1) // n
    return pl.pallas_call(
        _transpose_body,
        grid=(grid,),
        in_specs=[pl.BlockSpec((_D, n), lambda g: (0, g)),
                  pl.BlockSpec((_D, n), lambda g: (0, g))],
        out_specs=pl.BlockSpec((n, 2 * _D), lambda g: (g, 0)),
        out_shape=jax.ShapeDtypeStruct((v, 2 * _D), jnp.float32),
    )(ut, vt)


def _loss_body(pos_ref, neg_ref, out_ref):
    p = pos_ref[...]
    n = neg_ref[...]
    valid = lax.broadcasted_iota(jnp.int32, p.shape, 1) % 16 < _ICHUNK
    lp = jnp.minimum(p, 0.0) - jnp.log1p(jnp.exp(-jnp.abs(p)))
    lp = jnp.where(valid, lp, 0.0)
    ln = jnp.minimum(-n, 0.0) - jnp.log1p(jnp.exp(-jnp.abs(n)))
    out_ref[0, 0] = -(jnp.sum(lp) + jnp.sum(ln))


def kernel(pos_u, pos_v, neg_v, u_weight, v_weight):
    pu = pos_u.reshape(_NW, _NCHUNKS, _ICHUNK).astype(jnp.int32)
    pv = pos_v.reshape(_NW, _NCHUNKS, _ICHUNK).astype(jnp.int32)
    nv = neg_v.reshape(_NW, _NCHUNKS, _ROWS).astype(jnp.int32)
    idx = jnp.concatenate([pu, pv, nv], axis=2)

    cw = _combined_table(u_weight.T, v_weight.T)
    pos_s, neg_s = _sc_scores(idx, cw)

    loss = pl.pallas_call(
        _loss_body,
        out_shape=jax.ShapeDtypeStruct((1, 1), jnp.float32),
        out_specs=pl.BlockSpec(memory_space=pltpu.SMEM),
    )(pos_s.reshape(512, 128), neg_s.reshape(2560, 128))
    return loss[0, 0]


# speedup vs baseline: 2.6649x; 1.1152x over previous
import functools

import jax
import jax.numpy as jnp
from jax import lax
from jax.experimental import pallas as pl
from jax.experimental.pallas import tpu as pltpu
from jax.experimental.pallas import tpu_sc as plsc

_B = 16384
_K = 20
_D = 64
_NW = 32
_N_PER_W = _B // _NW
_ICHUNK = 4
_NCHUNKS = _N_PER_W // _ICHUNK
_ROWS = _ICHUNK * _K
_GROWS = 2 * _ICHUNK + _ROWS


def _sc_scores(idx_r, cw):
    mesh = plsc.VectorSubcoreMesh(core_axis_name="c", subcore_axis_name="s")

    @functools.partial(
        pl.kernel,
        mesh=mesh,
        compiler_params=pltpu.CompilerParams(use_tc_tiling_on_sc=False),
        out_type=[
            jax.ShapeDtypeStruct((_NW, _NCHUNKS * 16), jnp.float32),
            jax.ShapeDtypeStruct((_NW, _N_PER_W * _K), jnp.float32),
        ],
        scratch_types=[
            pltpu.VMEM((_NCHUNKS, _GROWS), jnp.int32),
            pltpu.VMEM((_GROWS, 2 * _D), jnp.float32),
            pltpu.VMEM((_GROWS, 2 * _D), jnp.float32),
            pltpu.VMEM((_NCHUNKS * 16,), jnp.float32),
            pltpu.VMEM((_N_PER_W * _K,), jnp.float32),
            pltpu.SemaphoreType.DMA,
            pltpu.SemaphoreType.DMA,
        ],
    )
    def scores_kernel(idx_hbm, cw_hbm, pos_out, neg_out,
                      idx_v, nb0, nb1, pos_v, neg_sv, sem0, sem1):
        wid = lax.axis_index("s") * 2 + lax.axis_index("c")
        lane = lax.iota(jnp.int32, 16)
        perms = [lane ^ sh for sh in (8, 4, 2, 1)]

        def xsum(x):
            for p in perms:
                x = x + x.at[p].get(mode="promise_in_bounds")
            return x

        pltpu.sync_copy(idx_hbm.at[wid], idx_v)

        def fire(c, buf, sem):
            pltpu.async_copy(cw_hbm.at[idx_v.at[c]], buf, sem)

        def wait(c, buf, sem):
            pltpu.make_async_copy(cw_hbm.at[idx_v.at[c]], buf, sem).wait()

        fire(0, nb0, sem0)
        fire(1, nb1, sem1)

        def compute_chunk(c, buf):
            rpos = jnp.zeros((16,), jnp.float32)
            rneg = [jnp.zeros((16,), jnp.float32) for _ in range(5)]
            for ii in range(_ICHUNK):
                u = [buf[ii, pl.ds(16 * j, 16)] for j in range(4)]
                v = [buf[4 + ii, pl.ds(_D + 16 * j, 16)] for j in range(4)]
                s = u[0] * v[0] + u[1] * v[1] + u[2] * v[2] + u[3] * v[3]
                rpos = jnp.where(lane == ii, xsum(s), rpos)
                for k in range(_K):
                    g = ii * _K + k
                    n = [buf[8 + g, pl.ds(_D + 16 * j, 16)] for j in range(4)]
                    s = u[0] * n[0] + u[1] * n[1] + u[2] * n[2] + u[3] * n[3]
                    rneg[g // 16] = jnp.where(lane == (g % 16), xsum(s),
                                              rneg[g // 16])
            pos_v[pl.ds(c * 16, 16)] = rpos
            for j in range(5):
                neg_sv[pl.ds(c * _ROWS + j * 16, 16)] = rneg[j]

        def body(t, carry):
            c0 = 2 * t
            wait(c0, nb0, sem0)

            @pl.when(t <= (_NCHUNKS // 2) - 2)
            def _():
                fire(c0 + 2, nb0, sem0)

            compute_chunk(c0, nb0)

            c1 = 2 * t + 1
            wait(c1, nb1, sem1)

            @pl.when(t <= (_NCHUNKS // 2) - 2)
            def _():
                fire(c1 + 2, nb1, sem1)

            compute_chunk(c1, nb1)
            return carry

        lax.fori_loop(0, _NCHUNKS // 2, body, 0)

        pltpu.sync_copy(pos_v, pos_out.at[wid])
        pltpu.sync_copy(neg_sv, neg_out.at[wid])

    return scores_kernel(idx_r, cw)


def _transpose_body(u_ref, v_ref, out_ref):
    out_ref[:, :_D] = u_ref[...].T
    out_ref[:, _D:] = v_ref[...].T


def _combined_table(ut, vt):
    v = ut.shape[1]
    n = 8192
    grid = (v + n - 1) // n
    return pl.pallas_call(
        _transpose_body,
        grid=(grid,),
        in_specs=[pl.BlockSpec((_D, n), lambda g: (0, g)),
                  pl.BlockSpec((_D, n), lambda g: (0, g))],
        out_specs=pl.BlockSpec((n, 2 * _D), lambda g: (g, 0)),
        out_shape=jax.ShapeDtypeStruct((v, 2 * _D), jnp.float32),
    )(ut, vt)


def _loss_body(pos_ref, neg_ref, out_ref):
    p = pos_ref[...]
    n = neg_ref[...]
    valid = lax.broadcasted_iota(jnp.int32, p.shape, 1) % 16 < _ICHUNK
    lp = jnp.minimum(p, 0.0) - jnp.log1p(jnp.exp(-jnp.abs(p)))
    lp = jnp.where(valid, lp, 0.0)
    ln = jnp.minimum(-n, 0.0) - jnp.log1p(jnp.exp(-jnp.abs(n)))
    out_ref[0, 0] = -(jnp.sum(lp) + jnp.sum(ln))


def kernel(pos_u, pos_v, neg_v, u_weight, v_weight):
    pu = pos_u.reshape(_NW, _NCHUNKS, _ICHUNK).astype(jnp.int32)
    pv = pos_v.reshape(_NW, _NCHUNKS, _ICHUNK).astype(jnp.int32)
    nv = neg_v.reshape(_NW, _NCHUNKS, _ROWS).astype(jnp.int32)
    idx = jnp.concatenate([pu, pv, nv], axis=2)

    cw = _combined_table(u_weight.T, v_weight.T)
    pos_s, neg_s = _sc_scores(idx, cw)

    loss = pl.pallas_call(
        _loss_body,
        out_shape=jax.ShapeDtypeStruct((1, 1), jnp.float32),
        out_specs=pl.BlockSpec(memory_space=pltpu.SMEM),
    )(pos_s.reshape(512, 128), neg_s.reshape(2560, 128))
    return loss[0, 0]


# speedup vs baseline: 2.8183x; 1.0576x over previous
import functools

import jax
import jax.numpy as jnp
from jax import lax
from jax.experimental import pallas as pl
from jax.experimental.pallas import tpu as pltpu
from jax.experimental.pallas import tpu_sc as plsc

_B = 16384
_K = 20
_D = 64
_NW = 32
_N_PER_W = _B // _NW
_ICHUNK = 4
_NCHUNKS = _N_PER_W // _ICHUNK
_ROWS = _ICHUNK * _K
_GROWS = 2 * _ICHUNK + _ROWS


def _sc_scores(idx_r, cw):
    mesh = plsc.VectorSubcoreMesh(core_axis_name="c", subcore_axis_name="s")

    @functools.partial(
        pl.kernel,
        mesh=mesh,
        compiler_params=pltpu.CompilerParams(use_tc_tiling_on_sc=False),
        out_type=[
            jax.ShapeDtypeStruct((_NW, _NCHUNKS * 16), jnp.float32),
            jax.ShapeDtypeStruct((_NW, _N_PER_W * _K), jnp.float32),
        ],
        scratch_types=[
            pltpu.VMEM((_NCHUNKS, _GROWS), jnp.int32),
            pltpu.VMEM((_GROWS, 2 * _D), jnp.float32),
            pltpu.VMEM((_GROWS, 2 * _D), jnp.float32),
            pltpu.VMEM((_NCHUNKS * 16,), jnp.float32),
            pltpu.VMEM((_N_PER_W * _K,), jnp.float32),
            pltpu.SemaphoreType.DMA,
            pltpu.SemaphoreType.DMA,
        ],
    )
    def scores_kernel(idx_hbm, cw_hbm, pos_out, neg_out,
                      idx_v, nb0, nb1, pos_v, neg_sv, sem0, sem1):
        wid = lax.axis_index("s") * 2 + lax.axis_index("c")
        lane = lax.iota(jnp.int32, 16)
        perms = [lane ^ sh for sh in (8, 4, 2, 1)]

        def xsum(x):
            for p in perms:
                x = x + x.at[p].get(mode="promise_in_bounds")
            return x

        pltpu.sync_copy(idx_hbm.at[wid], idx_v)

        def fire(c, buf, sem):
            pltpu.async_copy(cw_hbm.at[idx_v.at[c]], buf, sem)

        def wait(c, buf, sem):
            pltpu.make_async_copy(cw_hbm.at[idx_v.at[c]], buf, sem).wait()

        fire(0, nb0, sem0)
        fire(1, nb1, sem1)

        def compute_chunk(c, buf):
            rpos = jnp.zeros((16,), jnp.float32)
            rneg = [jnp.zeros((16,), jnp.float32) for _ in range(5)]
            for ii in range(_ICHUNK):
                u = [buf[ii, pl.ds(16 * j, 16)] for j in range(4)]
                v = [buf[4 + ii, pl.ds(_D + 16 * j, 16)] for j in range(4)]
                s = u[0] * v[0] + u[1] * v[1] + u[2] * v[2] + u[3] * v[3]
                rpos = jnp.where(lane == ii, xsum(s), rpos)
                for k in range(_K):
                    g = ii * _K + k
                    n = [buf[8 + g, pl.ds(_D + 16 * j, 16)] for j in range(4)]
                    s = u[0] * n[0] + u[1] * n[1] + u[2] * n[2] + u[3] * n[3]
                    rneg[g // 16] = jnp.where(lane == (g % 16), xsum(s),
                                              rneg[g // 16])
            pos_v[pl.ds(c * 16, 16)] = rpos
            for j in range(5):
                neg_sv[pl.ds(c * _ROWS + j * 16, 16)] = rneg[j]

        def body(t, carry):
            c0 = 2 * t
            wait(c0, nb0, sem0)

            @pl.when(t <= (_NCHUNKS // 2) - 2)
            def _():
                fire(c0 + 2, nb0, sem0)

            compute_chunk(c0, nb0)

            c1 = 2 * t + 1
            wait(c1, nb1, sem1)

            @pl.when(t <= (_NCHUNKS // 2) - 2)
            def _():
                fire(c1 + 2, nb1, sem1)

            compute_chunk(c1, nb1)
            return carry

        lax.fori_loop(0, _NCHUNKS // 2, body, 0)

        pltpu.sync_copy(pos_v, pos_out.at[wid])
        pltpu.sync_copy(neg_sv, neg_out.at[wid])

    return scores_kernel(idx_r, cw)


def _transpose_body(u_ref, v_ref, out_ref):
    out_ref[:, :_D] = u_ref[...].T
    out_ref[:, _D:] = v_ref[...].T


def _combined_table(ut, vt):
    v = ut.shape[1]
    n = 16384
    grid = (v + n - 1) // n
    return pl.pallas_call(
        _transpose_body,
        grid=(grid,),
        in_specs=[pl.BlockSpec((_D, n), lambda g: (0, g)),
                  pl.BlockSpec((_D, n), lambda g: (0, g))],
        out_specs=pl.BlockSpec((n, 2 * _D), lambda g: (g, 0)),
        out_shape=jax.ShapeDtypeStruct((v, 2 * _D), jnp.float32),
    )(ut, vt)


def _loss_body(pos_ref, neg_ref, out_ref):
    p = pos_ref[...]
    n = neg_ref[...]
    valid = lax.broadcasted_iota(jnp.int32, p.shape, 1) % 16 < _ICHUNK
    lp = jnp.minimum(p, 0.0) - jnp.log1p(jnp.exp(-jnp.abs(p)))
    lp = jnp.where(valid, lp, 0.0)
    ln = jnp.minimum(-n, 0.0) - jnp.log1p(jnp.exp(-jnp.abs(n)))
    out_ref[0, 0] = -(jnp.sum(lp) + jnp.sum(ln))


def kernel(pos_u, pos_v, neg_v, u_weight, v_weight):
    pu = pos_u.reshape(_NW, _NCHUNKS, _ICHUNK).astype(jnp.int32)
    pv = pos_v.reshape(_NW, _NCHUNKS, _ICHUNK).astype(jnp.int32)
    nv = neg_v.reshape(_NW, _NCHUNKS, _ROWS).astype(jnp.int32)
    idx = jnp.concatenate([pu, pv, nv], axis=2)

    cw = _combined_table(u_weight.T, v_weight.T)
    pos_s, neg_s = _sc_scores(idx, cw)

    loss = pl.pallas_call(
        _loss_body,
        out_shape=jax.ShapeDtypeStruct((1, 1), jnp.float32),
        out_specs=pl.BlockSpec(memory_space=pltpu.SMEM),
    )(pos_s.reshape(512, 128), neg_s.reshape(2560, 128))
    return loss[0, 0]


# speedup vs baseline: 2.8388x; 1.0073x over previous
import functools

import jax
import jax.numpy as jnp
from jax import lax
from jax.experimental import pallas as pl
from jax.experimental.pallas import tpu as pltpu
from jax.experimental.pallas import tpu_sc as plsc

_B = 16384
_K = 20
_D = 64
_NW = 32
_N_PER_W = _B // _NW
_ICHUNK = 4
_NCHUNKS = _N_PER_W // _ICHUNK
_ROWS = _ICHUNK * _K
_GROWS = 2 * _ICHUNK + _ROWS


def _sc_scores(idx_r, cw):
    mesh = plsc.VectorSubcoreMesh(core_axis_name="c", subcore_axis_name="s")

    @functools.partial(
        pl.kernel,
        mesh=mesh,
        compiler_params=pltpu.CompilerParams(use_tc_tiling_on_sc=False),
        out_type=[
            jax.ShapeDtypeStruct((_NW, _NCHUNKS * 16), jnp.float32),
            jax.ShapeDtypeStruct((_NW, _N_PER_W * _K), jnp.float32),
        ],
        scratch_types=[
            pltpu.VMEM((_NCHUNKS, _GROWS), jnp.int32),
            pltpu.VMEM((_GROWS, 2 * _D), jnp.float32),
            pltpu.VMEM((_GROWS, 2 * _D), jnp.float32),
            pltpu.VMEM((_NCHUNKS * 16,), jnp.float32),
            pltpu.VMEM((_N_PER_W * _K,), jnp.float32),
            pltpu.SemaphoreType.DMA,
            pltpu.SemaphoreType.DMA,
        ],
    )
    def scores_kernel(idx_hbm, cw_hbm, pos_out, neg_out,
                      idx_v, nb0, nb1, pos_v, neg_sv, sem0, sem1):
        wid = lax.axis_index("s") * 2 + lax.axis_index("c")
        lane = lax.iota(jnp.int32, 16)
        perms = [lane ^ sh for sh in (8, 4, 2, 1)]

        def xsum(x):
            for p in perms:
                x = x + x.at[p].get(mode="promise_in_bounds")
            return x

        pltpu.sync_copy(idx_hbm.at[wid], idx_v)

        def fire(c, buf, sem):
            pltpu.async_copy(cw_hbm.at[idx_v.at[c]], buf, sem)

        def wait(c, buf, sem):
            pltpu.make_async_copy(cw_hbm.at[idx_v.at[c]], buf, sem).wait()

        fire(0, nb0, sem0)
        fire(1, nb1, sem1)

        def compute_chunk(c, buf):
            rpos = jnp.zeros((16,), jnp.float32)
            rneg = [jnp.zeros((16,), jnp.float32) for _ in range(5)]
            for ii in range(_ICHUNK):
                u = [buf[ii, pl.ds(16 * j, 16)] for j in range(4)]
                v = [buf[4 + ii, pl.ds(_D + 16 * j, 16)] for j in range(4)]
                s = u[0] * v[0] + u[1] * v[1] + u[2] * v[2] + u[3] * v[3]
                rpos = jnp.where(lane == ii, xsum(s), rpos)
                for k in range(_K):
                    g = ii * _K + k
                    n = [buf[8 + g, pl.ds(_D + 16 * j, 16)] for j in range(4)]
                    s = u[0] * n[0] + u[1] * n[1] + u[2] * n[2] + u[3] * n[3]
                    rneg[g // 16] = jnp.where(lane == (g % 16), xsum(s),
                                              rneg[g // 16])
            pos_v[pl.ds(c * 16, 16)] = rpos
            for j in range(5):
                neg_sv[pl.ds(c * _ROWS + j * 16, 16)] = rneg[j]

        def body(t, carry):
            c0 = 2 * t
            wait(c0, nb0, sem0)

            @pl.when(t <= (_NCHUNKS // 2) - 2)
            def _():
                fire(c0 + 2, nb0, sem0)

            compute_chunk(c0, nb0)

            c1 = 2 * t + 1
            wait(c1, nb1, sem1)

            @pl.when(t <= (_NCHUNKS // 2) - 2)
            def _():
                fire(c1 + 2, nb1, sem1)

            compute_chunk(c1, nb1)
            return carry

        lax.fori_loop(0, _NCHUNKS // 2, body, 0)

        pltpu.sync_copy(pos_v, pos_out.at[wid])
        pltpu.sync_copy(neg_sv, neg_out.at[wid])

    return scores_kernel(idx_r, cw)


def _transpose_body(u_ref, v_ref, out_ref):
    out_ref[:, :_D] = u_ref[...].T
    out_ref[:, _D:] = v_ref[...].T


def _combined_table(ut, vt):
    v = ut.shape[1]
    n = 20480
    grid = (v + n - 1) // n
    return pl.pallas_call(
        _transpose_body,
        grid=(grid,),
        in_specs=[pl.BlockSpec((_D, n), lambda g: (0, g)),
                  pl.BlockSpec((_D, n), lambda g: (0, g))],
        out_specs=pl.BlockSpec((n, 2 * _D), lambda g: (g, 0)),
        out_shape=jax.ShapeDtypeStruct((v, 2 * _D), jnp.float32),
    )(ut, vt)


def _loss_body(pos_ref, neg_ref, out_ref):
    p = pos_ref[...]
    n = neg_ref[...]
    valid = lax.broadcasted_iota(jnp.int32, p.shape, 1) % 16 < _ICHUNK
    lp = jnp.minimum(p, 0.0) - jnp.log1p(jnp.exp(-jnp.abs(p)))
    lp = jnp.where(valid, lp, 0.0)
    ln = jnp.minimum(-n, 0.0) - jnp.log1p(jnp.exp(-jnp.abs(n)))
    out_ref[0, 0] = -(jnp.sum(lp) + jnp.sum(ln))


def kernel(pos_u, pos_v, neg_v, u_weight, v_weight):
    pu = pos_u.reshape(_NW, _NCHUNKS, _ICHUNK).astype(jnp.int32)
    pv = pos_v.reshape(_NW, _NCHUNKS, _ICHUNK).astype(jnp.int32)
    nv = neg_v.reshape(_NW, _NCHUNKS, _ROWS).astype(jnp.int32)
    idx = jnp.concatenate([pu, pv, nv], axis=2)

    cw = _combined_table(u_weight.T, v_weight.T)
    pos_s, neg_s = _sc_scores(idx, cw)

    loss = pl.pallas_call(
        _loss_body,
        out_shape=jax.ShapeDtypeStruct((1, 1), jnp.float32),
        out_specs=pl.BlockSpec(memory_space=pltpu.SMEM),
    )(pos_s.reshape(512, 128), neg_s.reshape(2560, 128))
    return loss[0, 0]
